# baseline (speedup 1.0000x reference)
import math

import jax
import jax.numpy as jnp
from jax.experimental import pallas as pl

G = 512
A = 16
N = G * A
TYPE_DIM = 100
TIME_DIM = 128
H = 128
L = 4
NFREQ = 10
GB = 64


def _silu(x):
    return x * jax.nn.sigmoid(x)


def _ln(x, g, b):
    m = jnp.mean(x, axis=-1, keepdims=True)
    xc = x - m
    v = jnp.mean(xc * xc, axis=-1, keepdims=True)
    return xc * jax.lax.rsqrt(v + 1e-5) * g + b


def _dot(x, w):
    return jax.lax.dot_general(
        x, w, (((x.ndim - 1,), (0,)), ((), ())),
        preferred_element_type=jnp.float32)


def _gnn_block(t_ref, at_ref, fc_ref, lp_ref, tfreq_ref, sfreq_ref,
               WtsT_ref, bts_ref, WneA_ref, WneB_ref, bne_ref,
               mW1a_ref, mW1b_ref, mW1s_ref, mW1c_ref, mW1lp_ref, mb1_ref,
               mW2_ref, mb2_ref,
               aW1a_ref, aW1b_ref, ab1_ref, aW2_ref, ab2_ref,
               lng_ref, lnb_ref, flng_ref, flnb_ref,
               WtrT_ref, btr_ref, WlpT_ref, WfcT_ref,
               type_out, lpp_out, fcp_out):
    nb = GB * A
    E = GB * A * A
    inv_a = 1.0 / A

    temb = _dot(at_ref[...], WtsT_ref[...]) + bts_ref[...]
    targ = t_ref[...] * tfreq_ref[...]
    temb_t = jnp.concatenate([jnp.sin(targ), jnp.cos(targ)], axis=1)
    tproj = _dot(temb_t, WneB_ref[...])
    nf = _dot(temb, WneA_ref[...]) + bne_ref[...]
    nf = (nf.reshape(GB, A, H) + tproj[:, None, :]).reshape(nb, H)

    narg = _dot(fc_ref[...], sfreq_ref[...])
    ns = jnp.sin(narg)
    nc = jnp.cos(narg)
    nsj = ns.reshape(GB, 1, A, 3 * NFREQ)
    ncj = nc.reshape(GB, 1, A, 3 * NFREQ)
    nsi = ns.reshape(GB, A, 1, 3 * NFREQ)
    nci = nc.reshape(GB, A, 1, 3 * NFREQ)
    fsin = (nsj * nci - ncj * nsi).reshape(E, 3 * NFREQ)
    fcos = (ncj * nci + nsj * nsi).reshape(E, 3 * NFREQ)
    lp = lp_ref[...]

    for l in range(L):
        nfn = _ln(nf, lng_ref[l], lnb_ref[l])
        asrc = _dot(nfn, mW1a_ref[l])
        bdst = _dot(nfn, mW1b_ref[l])
        cef = _dot(fsin, mW1s_ref[l]) + _dot(fcos, mW1c_ref[l])
        lpp = _dot(lp, mW1lp_ref[l]) + mb1_ref[l]
        pre = (cef.reshape(GB, A, A, H)
               + asrc.reshape(GB, A, 1, H)
               + bdst.reshape(GB, 1, A, H)
               + lpp.reshape(GB, 1, 1, H))
        h = _dot(_silu(pre).reshape(E, H), mW2_ref[l]) + mb2_ref[l]
        mij = _silu(h)
        msg = jnp.sum(mij.reshape(nb, A, H), axis=1) * inv_a
        agg = _dot(nf, aW1a_ref[l]) + _dot(msg, aW1b_ref[l]) + ab1_ref[l]
        agg = _silu(_dot(_silu(agg), aW2_ref[l]) + ab2_ref[l])
        nf = nf + agg

    nff = _ln(nf, flng_ref[...], flnb_ref[...])
    gf = jnp.sum(nff.reshape(GB, A, H), axis=1) * inv_a
    type_out[...] = _dot(nff, WtrT_ref[...]) + btr_ref[...]
    lpp_out[...] = _dot(gf, WlpT_ref[...])
    fcp_out[...] = _dot(nff, WfcT_ref[...])


def kernel(t, num_atoms, atom_types, frac_coords, l_polar, node2graph,
           W_ts, b_ts, W_ne, b_ne, msg_W1, msg_b1, msg_W2, msg_b2,
           agg_W1, agg_b1, agg_W2, agg_b2, ln_g, ln_b, fln_g, fln_b,
           W_tr, b_tr, W_lp, W_fc):
    f32 = jnp.float32
    half = TIME_DIM // 2
    tfreq = jnp.exp(
        jnp.arange(half, dtype=f32) * (-(math.log(10000.0) / (half - 1)))
    ).reshape(1, half)
    sfreq = jnp.kron(
        jnp.eye(3, dtype=f32),
        (2.0 * math.pi * jnp.arange(NFREQ, dtype=f32)).reshape(1, NFREQ))
    t2 = t.reshape(G, 1)

    WtsT = W_ts.T
    bts = b_ts.reshape(1, H)
    WneA = W_ne[:, :H].T
    WneB = W_ne[:, H:].T
    bne = b_ne.reshape(1, H)
    tr = lambda w: jnp.transpose(w, (0, 2, 1))
    mW1a = tr(msg_W1[:, :, :H])
    mW1b = tr(msg_W1[:, :, H:2 * H])
    mW1s = tr(msg_W1[:, :, 2 * H:2 * H + 3 * NFREQ])
    mW1c = tr(msg_W1[:, :, 2 * H + 3 * NFREQ:2 * H + 6 * NFREQ])
    mW1lp = tr(msg_W1[:, :, 2 * H + 6 * NFREQ:])
    mb1 = msg_b1.reshape(L, 1, H)
    mW2 = tr(msg_W2)
    mb2 = msg_b2.reshape(L, 1, H)
    aW1a = tr(agg_W1[:, :, :H])
    aW1b = tr(agg_W1[:, :, H:])
    ab1 = agg_b1.reshape(L, 1, H)
    aW2 = tr(agg_W2)
    ab2 = agg_b2.reshape(L, 1, H)
    lng = ln_g.reshape(L, 1, H)
    lnb = ln_b.reshape(L, 1, H)
    flng = fln_g.reshape(1, H)
    flnb = fln_b.reshape(1, H)
    WtrT = W_tr.T
    btr = b_tr.reshape(1, TYPE_DIM)
    WlpT = W_lp.T
    WfcT = W_fc.T

    nb = GB * A

    def full(shape):
        return pl.BlockSpec(shape, lambda i: tuple(0 for _ in shape))

    def node_bs(d):
        return pl.BlockSpec((nb, d), lambda i: (i, 0))

    def graph_bs(d):
        return pl.BlockSpec((GB, d), lambda i: (i, 0))

    out_shapes = (
        jax.ShapeDtypeStruct((N, TYPE_DIM), f32),
        jax.ShapeDtypeStruct((G, 6), f32),
        jax.ShapeDtypeStruct((N, 3), f32),
    )
    out_specs = (node_bs(TYPE_DIM), graph_bs(6), node_bs(3))

    return pl.pallas_call(
        _gnn_block,
        grid=(G // GB,),
        in_specs=[graph_bs(1), node_bs(TYPE_DIM), node_bs(3), graph_bs(6),
                  full((1, half)), full((3, 3 * NFREQ)),
                  full((TYPE_DIM, H)), full((1, H)), full((H, H)),
                  full((H, H)), full((1, H)),
                  full((L, H, H)), full((L, H, H)), full((L, 3 * NFREQ, H)),
                  full((L, 3 * NFREQ, H)), full((L, 6, H)), full((L, 1, H)),
                  full((L, H, H)), full((L, 1, H)),
                  full((L, H, H)), full((L, H, H)), full((L, 1, H)),
                  full((L, H, H)), full((L, 1, H)),
                  full((L, 1, H)), full((L, 1, H)), full((1, H)),
                  full((1, H)),
                  full((H, TYPE_DIM)), full((1, TYPE_DIM)), full((H, 6)),
                  full((H, 3))],
        out_specs=out_specs,
        out_shape=out_shapes,
    )(t2, atom_types, frac_coords, l_polar, tfreq, sfreq,
      WtsT, bts, WneA, WneB, bne,
      mW1a, mW1b, mW1s, mW1c, mW1lp, mb1, mW2, mb2,
      aW1a, aW1b, ab1, aW2, ab2,
      lng, lnb, flng, flnb,
      WtrT, btr, WlpT, WfcT)


# speedup vs baseline: 1.2115x; 1.2115x over previous
import math

import jax
import jax.numpy as jnp
from jax.experimental import pallas as pl

G = 512
A = 16
N = G * A
TYPE_DIM = 100
TIME_DIM = 128
H = 128
L = 4
NFREQ = 10
GB = 32


def _silu(x):
    return x * (0.5 * jnp.tanh(0.5 * x) + 0.5)


def _ln(x, g, b):
    m = jnp.mean(x, axis=-1, keepdims=True)
    xc = x - m
    v = jnp.mean(xc * xc, axis=-1, keepdims=True)
    return xc * jax.lax.rsqrt(v + 1e-5) * g + b


def _dot(x, w):
    return jax.lax.dot_general(
        x, w, (((x.ndim - 1,), (0,)), ((), ())),
        preferred_element_type=jnp.float32)


def _gnn_block(t_ref, at_ref, fc_ref, lp_ref, tfreq_ref, sfreq_ref,
               WtsT_ref, bts_ref, WneA_ref, WneB_ref, bne_ref,
               mW1a_ref, mW1b_ref, mW1s_ref, mW1c_ref, mW1lp_ref, mb1_ref,
               mW2_ref, mb2_ref,
               aW1a_ref, aW1b_ref, ab1_ref, aW2_ref, ab2_ref,
               lng_ref, lnb_ref, flng_ref, flnb_ref,
               WtrT_ref, btr_ref, WlpT_ref, WfcT_ref,
               type_out, lpp_out, fcp_out):
    nb = GB * A
    E = GB * A * A
    inv_a = 1.0 / A

    temb = _dot(at_ref[...], WtsT_ref[...]) + bts_ref[...]
    targ = t_ref[...] * tfreq_ref[...]
    temb_t = jnp.concatenate([jnp.sin(targ), jnp.cos(targ)], axis=1)
    tproj = _dot(temb_t, WneB_ref[...])
    nf = _dot(temb, WneA_ref[...]) + bne_ref[...]
    nf = (nf.reshape(GB, A, H) + tproj[:, None, :]).reshape(nb, H)

    narg = _dot(fc_ref[...], sfreq_ref[...])
    ns = jnp.sin(narg)
    nc = jnp.cos(narg)
    nsj = ns.reshape(GB, 1, A, 3 * NFREQ)
    ncj = nc.reshape(GB, 1, A, 3 * NFREQ)
    nsi = ns.reshape(GB, A, 1, 3 * NFREQ)
    nci = nc.reshape(GB, A, 1, 3 * NFREQ)
    fsin = (nsj * nci - ncj * nsi).reshape(E, 3 * NFREQ)
    fcos = (ncj * nci + nsj * nsi).reshape(E, 3 * NFREQ)
    lp = lp_ref[...]

    for l in range(L):
        nfn = _ln(nf, lng_ref[l], lnb_ref[l])
        asrc = _dot(nfn, mW1a_ref[l])
        bdst = _dot(nfn, mW1b_ref[l])
        cef = _dot(fsin, mW1s_ref[l]) + _dot(fcos, mW1c_ref[l])
        lpp = _dot(lp, mW1lp_ref[l]) + mb1_ref[l]
        pre = (cef.reshape(GB, A, A, H)
               + asrc.reshape(GB, A, 1, H)
               + bdst.reshape(GB, 1, A, H)
               + lpp.reshape(GB, 1, 1, H))
        h = _dot(_silu(pre).reshape(E, H), mW2_ref[l]) + mb2_ref[l]
        mij = _silu(h)
        msg = jnp.sum(mij.reshape(nb, A, H), axis=1) * inv_a
        agg = _dot(nf, aW1a_ref[l]) + _dot(msg, aW1b_ref[l]) + ab1_ref[l]
        agg = _silu(_dot(_silu(agg), aW2_ref[l]) + ab2_ref[l])
        nf = nf + agg

    nff = _ln(nf, flng_ref[...], flnb_ref[...])
    gf = jnp.sum(nff.reshape(GB, A, H), axis=1) * inv_a
    type_out[...] = _dot(nff, WtrT_ref[...]) + btr_ref[...]
    lpp_out[...] = _dot(gf, WlpT_ref[...])
    fcp_out[...] = _dot(nff, WfcT_ref[...])


def kernel(t, num_atoms, atom_types, frac_coords, l_polar, node2graph,
           W_ts, b_ts, W_ne, b_ne, msg_W1, msg_b1, msg_W2, msg_b2,
           agg_W1, agg_b1, agg_W2, agg_b2, ln_g, ln_b, fln_g, fln_b,
           W_tr, b_tr, W_lp, W_fc):
    f32 = jnp.float32
    half = TIME_DIM // 2
    tfreq = jnp.exp(
        jnp.arange(half, dtype=f32) * (-(math.log(10000.0) / (half - 1)))
    ).reshape(1, half)
    sfreq = jnp.kron(
        jnp.eye(3, dtype=f32),
        (2.0 * math.pi * jnp.arange(NFREQ, dtype=f32)).reshape(1, NFREQ))
    t2 = t.reshape(G, 1)

    WtsT = W_ts.T
    bts = b_ts.reshape(1, H)
    WneA = W_ne[:, :H].T
    WneB = W_ne[:, H:].T
    bne = b_ne.reshape(1, H)
    tr = lambda w: jnp.transpose(w, (0, 2, 1))
    mW1a = tr(msg_W1[:, :, :H])
    mW1b = tr(msg_W1[:, :, H:2 * H])
    mW1s = tr(msg_W1[:, :, 2 * H:2 * H + 3 * NFREQ])
    mW1c = tr(msg_W1[:, :, 2 * H + 3 * NFREQ:2 * H + 6 * NFREQ])
    mW1lp = tr(msg_W1[:, :, 2 * H + 6 * NFREQ:])
    mb1 = msg_b1.reshape(L, 1, H)
    mW2 = tr(msg_W2)
    mb2 = msg_b2.reshape(L, 1, H)
    aW1a = tr(agg_W1[:, :, :H])
    aW1b = tr(agg_W1[:, :, H:])
    ab1 = agg_b1.reshape(L, 1, H)
    aW2 = tr(agg_W2)
    ab2 = agg_b2.reshape(L, 1, H)
    lng = ln_g.reshape(L, 1, H)
    lnb = ln_b.reshape(L, 1, H)
    flng = fln_g.reshape(1, H)
    flnb = fln_b.reshape(1, H)
    WtrT = W_tr.T
    btr = b_tr.reshape(1, TYPE_DIM)
    WlpT = W_lp.T
    WfcT = W_fc.T

    nb = GB * A

    def full(shape):
        return pl.BlockSpec(shape, lambda i: tuple(0 for _ in shape))

    def node_bs(d):
        return pl.BlockSpec((nb, d), lambda i: (i, 0))

    def graph_bs(d):
        return pl.BlockSpec((GB, d), lambda i: (i, 0))

    out_shapes = (
        jax.ShapeDtypeStruct((N, TYPE_DIM), f32),
        jax.ShapeDtypeStruct((G, 6), f32),
        jax.ShapeDtypeStruct((N, 3), f32),
    )
    out_specs = (node_bs(TYPE_DIM), graph_bs(6), node_bs(3))

    return pl.pallas_call(
        _gnn_block,
        grid=(G // GB,),
        in_specs=[graph_bs(1), node_bs(TYPE_DIM), node_bs(3), graph_bs(6),
                  full((1, half)), full((3, 3 * NFREQ)),
                  full((TYPE_DIM, H)), full((1, H)), full((H, H)),
                  full((H, H)), full((1, H)),
                  full((L, H, H)), full((L, H, H)), full((L, 3 * NFREQ, H)),
                  full((L, 3 * NFREQ, H)), full((L, 6, H)), full((L, 1, H)),
                  full((L, H, H)), full((L, 1, H)),
                  full((L, H, H)), full((L, H, H)), full((L, 1, H)),
                  full((L, H, H)), full((L, 1, H)),
                  full((L, 1, H)), full((L, 1, H)), full((1, H)),
                  full((1, H)),
                  full((H, TYPE_DIM)), full((1, TYPE_DIM)), full((H, 6)),
                  full((H, 3))],
        out_specs=out_specs,
        out_shape=out_shapes,
    )(t2, atom_types, frac_coords, l_polar, tfreq, sfreq,
      WtsT, bts, WneA, WneB, bne,
      mW1a, mW1b, mW1s, mW1c, mW1lp, mb1, mW2, mb2,
      aW1a, aW1b, ab1, aW2, ab2,
      lng, lnb, flng, flnb,
      WtrT, btr, WlpT, WfcT)
